# baseline (speedup 1.0000x reference)
import functools

import jax
import jax.numpy as jnp
from jax import lax
from jax.experimental import pallas as pl
from jax.experimental.pallas import tpu as pltpu
from jax.experimental.pallas import tpu_sc as plsc

HIDDEN = 64
OUT = 128
NW = 32
C = 128


def _tile_proj_body(tt, w, o):
    o[...] = jnp.dot(tt[...], w[...], preferred_element_type=jnp.float32)


def _tile_proj(tile_table, w_t):
    n = tile_table.shape[0]
    blk = 1000
    return pl.pallas_call(
        _tile_proj_body,
        grid=(n // blk,),
        in_specs=[pl.BlockSpec((blk, HIDDEN), lambda i: (i, 0)),
                  pl.BlockSpec((HIDDEN, OUT), lambda i: (0, 0))],
        out_specs=pl.BlockSpec((blk, OUT), lambda i: (i, 0)),
        out_shape=jax.ShapeDtypeStruct((n, OUT), jnp.float32),
    )(tile_table, w_t)


def _colrow_body(col, row, wc, wr, b, o):
    ce = jnp.dot(col[...], wc[...], preferred_element_type=jnp.float32)
    re = jnp.dot(row[...], wr[...], preferred_element_type=jnp.float32) + b[...]
    o[...] = ce[:, None, :] + re[None, :, :]


def _colrow_proj(col_table, row_table, wc, wr, b):
    wd, hd = col_table.shape[0], row_table.shape[0]
    blk = 40
    out = pl.pallas_call(
        _colrow_body,
        grid=(wd // blk,),
        in_specs=[pl.BlockSpec((blk, HIDDEN), lambda i: (i, 0)),
                  pl.BlockSpec((hd, HIDDEN), lambda i: (0, 0)),
                  pl.BlockSpec((HIDDEN, OUT), lambda i: (0, 0)),
                  pl.BlockSpec((HIDDEN, OUT), lambda i: (0, 0)),
                  pl.BlockSpec((1, OUT), lambda i: (0, 0))],
        out_specs=pl.BlockSpec((blk, hd, OUT), lambda i: (i, 0, 0)),
        out_shape=jax.ShapeDtypeStruct((wd, hd, OUT), jnp.float32),
    )(col_table, row_table, wc, wr, b.reshape(1, OUT))
    return out.reshape(wd * hd, OUT)


def _sc_body(chunks_pw, idx_hbm, tp_hbm, cr_hbm, out_hbm,
             ibuf, bt0, bt1, bc0, bc1,
             is0, is1, is2, is3, gt0, gt1, gc0, gc1, os0, os1):
    wid = lax.axis_index("s") * 2 + lax.axis_index("c")
    c0 = wid * chunks_pw
    bts, bcs = [bt0, bt1], [bc0, bc1]
    isems = [is0, is1, is2, is3]
    gts, gcs, oss = [gt0, gt1], [gc0, gc1], [os0, os1]

    def fire_idx(g, s):
        pltpu.async_copy(idx_hbm.at[c0 + g], ibuf.at[s], isems[s])

    def wait_idx(g, s):
        pltpu.make_async_copy(idx_hbm.at[c0 + g], ibuf.at[s], isems[s]).wait()

    def fire_gathers(g, s, d):
        pltpu.async_copy(tp_hbm.at[ibuf.at[s, 0]], bts[d], gts[d])
        pltpu.async_copy(cr_hbm.at[ibuf.at[s, 1]], bcs[d], gcs[d])

    def wait_out(g, d):
        pltpu.make_async_copy(
            bts[d], out_hbm.at[pl.ds((c0 + g) * C, C)], oss[d]).wait()

    def complete(g, s, d):
        pltpu.make_async_copy(tp_hbm.at[ibuf.at[s, 0]], bts[d], gts[d]).wait()
        pltpu.make_async_copy(cr_hbm.at[ibuf.at[s, 1]], bcs[d], gcs[d]).wait()
        bt, bc = bts[d], bcs[d]

        @plsc.parallel_loop(0, C, step=1, unroll=4)
        def rows(i):
            for k in range(OUT // 16):
                sl = pl.ds(k * 16, 16)
                plsc.addupdate(bt.at[i, sl], bc[i, sl])
        pltpu.async_copy(bt, out_hbm.at[pl.ds((c0 + g) * C, C)], oss[d])

    pltpu.sync_copy(idx_hbm.at[c0], ibuf.at[0])
    pltpu.sync_copy(idx_hbm.at[c0 + 1], ibuf.at[1])
    fire_gathers(0, 0, 0)
    fire_idx(2, 2)
    fire_gathers(1, 1, 1)
    fire_idx(3, 3)
    complete(0, 0, 0)
    wait_out(0, 0)
    wait_idx(2, 2)
    fire_gathers(2, 2, 0)
    fire_idx(4, 0)
    complete(1, 1, 1)
    wait_out(1, 1)
    wait_idx(3, 3)
    fire_gathers(3, 3, 1)
    fire_idx(5, 1)
    complete(2, 2, 0)

    def body(p, carry):
        for j in range(4):
            g = 4 * p + j
            d = j % 2
            wait_out(g - 2, d)
            wait_idx(g, j)
            fire_gathers(g, j, d)

            @pl.when(g + 2 < chunks_pw)
            def _():
                fire_idx(g + 2, (j + 2) % 4)

            complete(g - 1, (j - 1) % 4, 1 - d)
        return carry
    lax.fori_loop(1, chunks_pw // 4, body, 0)

    g_last = chunks_pw - 1
    complete(g_last, 3, 1)
    wait_out(g_last - 1, 0)
    wait_out(g_last, 1)


def _sc_call(idx3, tp, cr):
    nchunks = idx3.shape[0]
    tokens = nchunks * C
    mesh = plsc.VectorSubcoreMesh(core_axis_name="c", subcore_axis_name="s")
    kfn = pl.kernel(
        functools.partial(_sc_body, nchunks // NW),
        out_type=jax.ShapeDtypeStruct((tokens, OUT), jnp.float32),
        mesh=mesh,
        scratch_types=[
            pltpu.VMEM((4, 2, C), jnp.int32),
            pltpu.VMEM((C, OUT), jnp.float32),
            pltpu.VMEM((C, OUT), jnp.float32),
            pltpu.VMEM((C, OUT), jnp.float32),
            pltpu.VMEM((C, OUT), jnp.float32),
        ] + [pltpu.SemaphoreType.DMA] * 10,
    )
    return kfn(idx3, tp, cr)


def kernel(tile, x, y, tile_table, col_table, row_table, W, b):
    bsz, seq = tile.shape
    hd = row_table.shape[0]
    tp = _tile_proj(tile_table, W[:HIDDEN])
    cr = _colrow_proj(col_table, row_table, W[HIDDEN:2 * HIDDEN],
                      W[2 * HIDDEN:], b)
    nchunks = (bsz * seq) // C
    idx3 = jnp.stack([tile.reshape(nchunks, C),
                      (x * hd + y).reshape(nchunks, C).astype(jnp.int32)],
                     axis=1)
    out = _sc_call(idx3, tp, cr)
    return out.reshape(bsz, seq, OUT)


# speedup vs baseline: 1.0808x; 1.0808x over previous
import functools

import jax
import jax.numpy as jnp
from jax import lax
from jax.experimental import pallas as pl
from jax.experimental.pallas import tpu as pltpu
from jax.experimental.pallas import tpu_sc as plsc

HIDDEN = 64
OUT = 128
NW = 32
C = 128


def _tile_proj_body(tt, w, o):
    o[...] = jnp.dot(tt[...], w[...], preferred_element_type=jnp.float32)


def _tile_proj(tile_table, w_t):
    n = tile_table.shape[0]
    blk = 10000
    return pl.pallas_call(
        _tile_proj_body,
        grid=(n // blk,),
        in_specs=[pl.BlockSpec((blk, HIDDEN), lambda i: (i, 0)),
                  pl.BlockSpec((HIDDEN, OUT), lambda i: (0, 0))],
        out_specs=pl.BlockSpec((blk, OUT), lambda i: (i, 0)),
        out_shape=jax.ShapeDtypeStruct((n, OUT), jnp.float32),
    )(tile_table, w_t)


def _colrow_body(col, row, wc, wr, b, o):
    ce = jnp.dot(col[...], wc[...], preferred_element_type=jnp.float32)
    re = jnp.dot(row[...], wr[...], preferred_element_type=jnp.float32) + b[...]
    o[...] = ce[:, None, :] + re[None, :, :]


def _colrow_proj(col_table, row_table, wc, wr, b):
    wd, hd = col_table.shape[0], row_table.shape[0]
    blk = 40
    out = pl.pallas_call(
        _colrow_body,
        grid=(wd // blk,),
        in_specs=[pl.BlockSpec((blk, HIDDEN), lambda i: (i, 0)),
                  pl.BlockSpec((hd, HIDDEN), lambda i: (0, 0)),
                  pl.BlockSpec((HIDDEN, OUT), lambda i: (0, 0)),
                  pl.BlockSpec((HIDDEN, OUT), lambda i: (0, 0)),
                  pl.BlockSpec((1, OUT), lambda i: (0, 0))],
        out_specs=pl.BlockSpec((blk, hd, OUT), lambda i: (i, 0, 0)),
        out_shape=jax.ShapeDtypeStruct((wd, hd, OUT), jnp.float32),
    )(col_table, row_table, wc, wr, b.reshape(1, OUT))
    return out.reshape(wd * hd, OUT)


def _idx_body(hd, t, x, y, o):
    o[:, 0, :] = t[...]
    o[:, 1, :] = x[...] * hd + y[...]


def _idx_pack(tile_f, x_f, y_f, hd, nchunks):
    blk = 256
    return pl.pallas_call(
        functools.partial(_idx_body, hd),
        grid=(nchunks // blk,),
        in_specs=[pl.BlockSpec((blk, C), lambda i: (i, 0))] * 3,
        out_specs=pl.BlockSpec((blk, 2, C), lambda i: (i, 0, 0)),
        out_shape=jax.ShapeDtypeStruct((nchunks, 2, C), jnp.int32),
    )(tile_f.reshape(nchunks, C), x_f.reshape(nchunks, C),
      y_f.reshape(nchunks, C))


def _sc_body(chunks_pw, idx_hbm, tp_hbm, cr_hbm, out_hbm,
             ibuf, b0, b1, b2, b3, *sems):
    wid = lax.axis_index("s") * 2 + lax.axis_index("c")
    c0 = wid * chunks_pw
    bufs = [b0, b1, b2, b3]
    isems, tsems = sems[0:4], sems[4:8]
    csems, wsems = sems[8:12], sems[12:16]

    def w_copy(g, j):
        return pltpu.make_async_copy(
            bufs[j], out_hbm.at[pl.ds((c0 + g) * C, C)], wsems[j])

    for s in range(4):
        pltpu.async_copy(idx_hbm.at[c0 + s], ibuf.at[s], isems[s])

    def step(g, j):
        jm1, jm2 = (j - 1) % 4, (j - 2) % 4

        @pl.when(g >= 4)
        def _():
            w_copy(g - 4, j).wait()

        @pl.when(g < chunks_pw)
        def _():
            pltpu.make_async_copy(
                idx_hbm.at[c0 + g], ibuf.at[j], isems[j]).wait()
            pltpu.async_copy(tp_hbm.at[ibuf.at[j, 0]], bufs[j], tsems[j])

        @pl.when((g >= 1) & (g < chunks_pw + 1))
        def _():
            pltpu.make_async_copy(
                tp_hbm.at[ibuf.at[jm1, 0]], bufs[jm1], tsems[jm1]).wait()
            pltpu.async_copy(
                cr_hbm.at[ibuf.at[jm1, 1]], bufs[jm1], csems[jm1], add=True)

        @pl.when((g >= 2) & (g < chunks_pw + 2))
        def _():
            pltpu.make_async_copy(
                cr_hbm.at[ibuf.at[jm2, 1]], bufs[jm2], csems[jm2]).wait()
            pltpu.async_copy(
                bufs[jm2], out_hbm.at[pl.ds((c0 + g - 2) * C, C)], wsems[jm2])

            @pl.when(g + 2 < chunks_pw)
            def _():
                pltpu.async_copy(
                    idx_hbm.at[c0 + g + 2], ibuf.at[jm2], isems[jm2])

    def body(p, carry):
        for j in range(4):
            step(4 * p + j, j)
        return carry
    lax.fori_loop(0, (chunks_pw + 4) // 4, body, 0)


def _sc_call(idx3, tp, cr):
    nchunks = idx3.shape[0]
    tokens = nchunks * C
    mesh = plsc.VectorSubcoreMesh(core_axis_name="c", subcore_axis_name="s")
    kfn = pl.kernel(
        functools.partial(_sc_body, nchunks // NW),
        out_type=jax.ShapeDtypeStruct((tokens, OUT), jnp.float32),
        mesh=mesh,
        scratch_types=[
            pltpu.VMEM((4, 2, C), jnp.int32),
            pltpu.VMEM((C, OUT), jnp.float32),
            pltpu.VMEM((C, OUT), jnp.float32),
            pltpu.VMEM((C, OUT), jnp.float32),
            pltpu.VMEM((C, OUT), jnp.float32),
        ] + [pltpu.SemaphoreType.DMA] * 16,
    )
    return kfn(idx3, tp, cr)


def kernel(tile, x, y, tile_table, col_table, row_table, W, b):
    bsz, seq = tile.shape
    hd = row_table.shape[0]
    tp = _tile_proj(tile_table, W[:HIDDEN])
    cr = _colrow_proj(col_table, row_table, W[HIDDEN:2 * HIDDEN],
                      W[2 * HIDDEN:], b)
    nchunks = (bsz * seq) // C
    idx3 = _idx_pack(tile.reshape(-1), x.reshape(-1), y.reshape(-1),
                     hd, nchunks)
    out = _sc_call(idx3, tp, cr)
    return out.reshape(bsz, seq, OUT)
